# baseline (speedup 1.0000x reference)
import functools
import math

import jax
import jax.numpy as jnp
from jax import lax
from jax.experimental import pallas as pl
from jax.experimental.pallas import tpu as pltpu
from jax.experimental.pallas import tpu_sc as plsc

B, S, V, E, P, D, F, L, H, O = 1, 2048, 32000, 768, 256, 1024, 4096, 2, 16, 2048
DH = D // H
EPS = 1e-12
BM = 512


def _ln(x, g, b):
    m = jnp.mean(x, axis=-1, keepdims=True)
    v = jnp.mean((x - m) ** 2, axis=-1, keepdims=True)
    return (x - m) / jnp.sqrt(v + EPS) * g + b


def _dot(a, b):
    return jax.lax.dot_general(
        a, b, (((1,), (0,)), ((), ())), preferred_element_type=jnp.float32)


def _bf(x):
    return x.astype(jnp.bfloat16)


def _make_sc_gather():
    info = plsc.get_sparse_core_info()
    nc, ns = info.num_cores, info.num_subcores
    nw = nc * ns
    b_per_w = S // nw
    mesh = plsc.VectorSubcoreMesh(core_axis_name="c", subcore_axis_name="s")

    @functools.partial(
        pl.kernel, mesh=mesh,
        out_type=jax.ShapeDtypeStruct((S, E), jnp.float32),
        scratch_types=[
            pltpu.VMEM((b_per_w,), jnp.int32),
            pltpu.VMEM((b_per_w, E), jnp.float32),
            pltpu.SemaphoreType.DMA,
        ],
    )
    def k(table_hbm, idx_hbm, out_hbm, idx_v, rows_v, sem):
        wid = lax.axis_index("s") * nc + lax.axis_index("c")
        base = wid * b_per_w
        pltpu.sync_copy(idx_hbm.at[pl.ds(base, b_per_w)], idx_v)
        pltpu.async_copy(table_hbm.at[idx_v], rows_v, sem).wait()
        pltpu.sync_copy(rows_v, out_hbm.at[pl.ds(base, b_per_w)])

    return k


_sc_gather = None


def _gather_rows(emb, ids):
    global _sc_gather
    if _sc_gather is None:
        _sc_gather = _make_sc_gather()
    return _sc_gather(emb, ids)


def _qkv_body(x_ref, g_ref, b_ref, wq_ref, wk_ref, wv_ref,
              bq_ref, bk_ref, bv_ref, q_ref, k_ref, v_ref):
    xn = _bf(_ln(x_ref[...], g_ref[...], b_ref[...]))
    scale = 1.0 / math.sqrt(DH)
    q_ref[...] = _bf((_dot(xn, _bf(wq_ref[...])) + bq_ref[...]) * scale)
    k_ref[...] = _bf(_dot(xn, _bf(wk_ref[...])) + bk_ref[...])
    v_ref[...] = _bf(_dot(xn, _bf(wv_ref[...])) + bv_ref[...])


def _qkv(x, g, b, wq, wk, wv, bq, bk, bv):
    grid = (S // BM,)
    wspec = pl.BlockSpec((D, D), lambda i: (0, 0))
    bspec = pl.BlockSpec((1, D), lambda i: (0, 0))
    ospec = pl.BlockSpec((BM, D), lambda i: (i, 0))
    osh = jax.ShapeDtypeStruct((S, D), jnp.bfloat16)
    return pl.pallas_call(
        _qkv_body,
        grid=grid,
        in_specs=[pl.BlockSpec((BM, D), lambda i: (i, 0)),
                  bspec, bspec, wspec, wspec, wspec, bspec, bspec, bspec],
        out_specs=[ospec, ospec, ospec],
        out_shape=[osh, osh, osh],
        compiler_params=pltpu.CompilerParams(
            dimension_semantics=("parallel",)),
    )(x, g, b, wq, wk, wv, bq, bk, bv)


def _attn_body(q_ref, k_ref, v_ref, o_ref, *, hp):
    outs = []
    for h in range(hp):
        q = q_ref[:, h * DH:(h + 1) * DH]
        k = k_ref[:, h * DH:(h + 1) * DH]
        s = jax.lax.dot_general(
            q, k, (((1,), (1,)), ((), ())),
            preferred_element_type=jnp.float32)
        m = jnp.max(s, axis=-1, keepdims=True)
        e = jnp.exp(s - m)
        r = 1.0 / jnp.sum(e, axis=-1, keepdims=True)
        acc = _dot(_bf(e), v_ref[:, h * DH:(h + 1) * DH])
        outs.append(acc * r)
    o_ref[...] = _bf(jnp.concatenate(outs, axis=1))


def _attention(q, k, v_ext, bq_blk, hp=2):
    grid = (H // hp, S // bq_blk)
    return pl.pallas_call(
        functools.partial(_attn_body, hp=hp),
        grid=grid,
        in_specs=[
            pl.BlockSpec((bq_blk, hp * DH), lambda h, i: (i, h)),
            pl.BlockSpec((S, hp * DH), lambda h, i: (0, h)),
            pl.BlockSpec((S, hp * DH), lambda h, i: (0, h)),
        ],
        out_specs=pl.BlockSpec((bq_blk, hp * DH), lambda h, i: (i, h)),
        out_shape=jax.ShapeDtypeStruct((S, D), jnp.bfloat16),
        compiler_params=pltpu.CompilerParams(
            dimension_semantics=("parallel", "parallel")),
    )(q, k, v_ext)


def _matmul_res_body(x_ref, w_ref, bias_ref, r_ref, o_ref):
    o_ref[...] = _dot(x_ref[...], _bf(w_ref[...])) + bias_ref[...] + r_ref[...]


def _matmul_res(x, w, bias, res, bn):
    k, n = w.shape
    grid = (n // bn, S // BM)
    return pl.pallas_call(
        _matmul_res_body,
        grid=grid,
        in_specs=[
            pl.BlockSpec((BM, k), lambda j, i: (i, 0)),
            pl.BlockSpec((k, bn), lambda j, i: (0, j)),
            pl.BlockSpec((1, bn), lambda j, i: (0, j)),
            pl.BlockSpec((BM, bn), lambda j, i: (i, j)),
        ],
        out_specs=pl.BlockSpec((BM, bn), lambda j, i: (i, j)),
        out_shape=jax.ShapeDtypeStruct((S, n), jnp.float32),
        compiler_params=pltpu.CompilerParams(
            dimension_semantics=("parallel", "parallel")),
    )(x, w, bias, res)


BQA = 256


def _attnblk_body(q_ref, k_ref, v_ref, wo_ref, bo_ref, r_ref, o_ref, o_s):
    for h in range(H):
        qb = q_ref[:, h * DH:(h + 1) * DH]
        s = jax.lax.dot_general(
            qb, k_ref[:, h * DH:(h + 1) * DH], (((1,), (1,)), ((), ())),
            preferred_element_type=jnp.float32)
        m = jnp.max(s, axis=-1, keepdims=True)
        e = jnp.exp(s - m)
        r = 1.0 / jnp.sum(e, axis=-1, keepdims=True)
        o_s[:, h * DH:(h + 1) * DH] = _bf(
            _dot(_bf(e), v_ref[:, h * DH:(h + 1) * DH]) * r)
    o_ref[...] = _dot(o_s[...], _bf(wo_ref[...])) + bo_ref[...] + r_ref[...]


def _attn_block(q, k, v, wo, bov, res):
    grid = (S // BQA,)
    return pl.pallas_call(
        _attnblk_body,
        grid=grid,
        in_specs=[
            pl.BlockSpec((BQA, D), lambda qi: (qi, 0)),
            pl.BlockSpec((S, D), lambda qi: (0, 0)),
            pl.BlockSpec((S, D), lambda qi: (0, 0)),
            pl.BlockSpec((D, D), lambda qi: (0, 0)),
            pl.BlockSpec((1, D), lambda qi: (0, 0)),
            pl.BlockSpec((BQA, D), lambda qi: (qi, 0)),
        ],
        out_specs=pl.BlockSpec((BQA, D), lambda qi: (qi, 0)),
        out_shape=jax.ShapeDtypeStruct((S, D), jnp.float32),
        scratch_shapes=[pltpu.VMEM((BQA, D), jnp.bfloat16)],
        compiler_params=pltpu.CompilerParams(
            dimension_semantics=("parallel",)),
    )(q, k, v, wo, bov, res)


def _ffn_body(x_ref, g_ref, b_ref, w1_ref, b1_ref, w2_ref, b2_ref, o_ref,
              xn_ref):
    j = pl.program_id(0)

    @pl.when(j == 0)
    def _():
        xn_ref[...] = _bf(_ln(x_ref[...], g_ref[...], b_ref[...]))

    a = _dot(xn_ref[...], _bf(w1_ref[...])) + b1_ref[...]
    hb = _bf(0.5 * a * (1.0 + lax.erf(a * (1.0 / math.sqrt(2.0)))))
    contrib = _dot(hb, _bf(w2_ref[...]))

    @pl.when(j == 0)
    def _():
        o_ref[...] = contrib + b2_ref[...] + x_ref[...]

    @pl.when(j > 0)
    def _():
        o_ref[...] += contrib


def _ffn(x, g, b, w1, b1, w2, b2, bf):
    grid = (F // bf,)
    return pl.pallas_call(
        _ffn_body,
        grid=grid,
        in_specs=[
            pl.BlockSpec((S, D), lambda j: (0, 0)),
            pl.BlockSpec((1, D), lambda j: (0, 0)),
            pl.BlockSpec((1, D), lambda j: (0, 0)),
            pl.BlockSpec((D, bf), lambda j: (0, j)),
            pl.BlockSpec((1, bf), lambda j: (0, j)),
            pl.BlockSpec((bf, D), lambda j: (j, 0)),
            pl.BlockSpec((1, D), lambda j: (0, 0)),
        ],
        out_specs=pl.BlockSpec((S, D), lambda j: (0, 0)),
        out_shape=jax.ShapeDtypeStruct((S, D), jnp.float32),
        scratch_shapes=[pltpu.VMEM((S, D), jnp.bfloat16)],
        compiler_params=pltpu.CompilerParams(
            dimension_semantics=("arbitrary",)),
    )(x, g, b, w1, b1, w2, b2)


def _ln_matmul_body(x_ref, g_ref, b_ref, w_ref, bias_ref, o_ref, *, gelu):
    xn = _bf(_ln(x_ref[...], g_ref[...], b_ref[...]))
    acc = _dot(xn, _bf(w_ref[...])) + bias_ref[...]
    if gelu:
        acc = 0.5 * acc * (1.0 + lax.erf(acc * (1.0 / math.sqrt(2.0))))
    o_ref[...] = acc.astype(o_ref.dtype)


def _ln_matmul(x, g, b, w, bias, bn, gelu=False):
    k, n = w.shape
    grid = (n // bn, S // BM)
    return pl.pallas_call(
        functools.partial(_ln_matmul_body, gelu=gelu),
        grid=grid,
        in_specs=[
            pl.BlockSpec((BM, k), lambda j, i: (i, 0)),
            pl.BlockSpec((1, k), lambda j, i: (0, 0)),
            pl.BlockSpec((1, k), lambda j, i: (0, 0)),
            pl.BlockSpec((k, bn), lambda j, i: (0, j)),
            pl.BlockSpec((1, bn), lambda j, i: (0, j)),
        ],
        out_specs=pl.BlockSpec((BM, bn), lambda j, i: (i, j)),
        out_shape=jax.ShapeDtypeStruct((S, n), jnp.bfloat16),
        compiler_params=pltpu.CompilerParams(
            dimension_semantics=("parallel", "parallel")),
    )(x, g, b, w, bias)


def _final_body(x_ref, w_ref, b_ref, o_ref):
    logits = _dot(_bf(x_ref[...]), _bf(w_ref[...])) + b_ref[...]
    m = jnp.max(logits, axis=-1, keepdims=True)
    e = jnp.exp(logits - m)
    o_ref[...] = e / jnp.sum(e, axis=-1, keepdims=True)


def _final(x, w, bias):
    grid = (S // BM,)
    return pl.pallas_call(
        _final_body,
        grid=grid,
        in_specs=[
            pl.BlockSpec((BM, D), lambda i: (i, 0)),
            pl.BlockSpec((D, O), lambda i: (0, 0)),
            pl.BlockSpec((1, O), lambda i: (0, 0)),
        ],
        out_specs=pl.BlockSpec((BM, O), lambda i: (i, 0)),
        out_shape=jax.ShapeDtypeStruct((S, O), jnp.float32),
        compiler_params=pltpu.CompilerParams(
            dimension_semantics=("parallel",)),
    )(x, w, bias)


def kernel(input_ids, probs, emb, ln1_g, ln1_b, Wq, bq, Wk, bk, Wv, bv,
           Wo, bo, ln2_g, ln2_b, W1, b1, W2, b2, Wlast, blast):
    rows = _gather_rows(emb, input_ids.reshape(S))
    x = jnp.concatenate([rows, probs.reshape(S, P)], axis=-1)

    for i in range(L):
        q, k, v = _qkv(x, ln1_g[i].reshape(1, D), ln1_b[i].reshape(1, D),
                       Wq[i], Wk[i], Wv[i], bq[i].reshape(1, D),
                       bk[i].reshape(1, D), bv[i].reshape(1, D))
        x = _attn_block(q, k, v, Wo[i], bo[i].reshape(1, D), x)
        x = _ffn(x, ln2_g[i].reshape(1, D), ln2_b[i].reshape(1, D),
                 W1[i], b1[i].reshape(1, F), W2[i], b2[i].reshape(1, D),
                 bf=1024)

    out = _final(x, Wlast, blast.reshape(1, O))
    return out.reshape(B, S, O)


# speedup vs baseline: 1.1014x; 1.1014x over previous
import functools
import math

import jax
import jax.numpy as jnp
from jax import lax
from jax.experimental import pallas as pl
from jax.experimental.pallas import tpu as pltpu
from jax.experimental.pallas import tpu_sc as plsc

B, S, V, E, P, D, F, L, H, O = 1, 2048, 32000, 768, 256, 1024, 4096, 2, 16, 2048
DH = D // H
EPS = 1e-12
BM = 512


def _ln(x, g, b):
    m = jnp.mean(x, axis=-1, keepdims=True)
    v = jnp.mean((x - m) ** 2, axis=-1, keepdims=True)
    return (x - m) / jnp.sqrt(v + EPS) * g + b


def _dot(a, b):
    return jax.lax.dot_general(
        a, b, (((1,), (0,)), ((), ())), preferred_element_type=jnp.float32)


def _bf(x):
    return x.astype(jnp.bfloat16)


def _make_sc_gather():
    info = plsc.get_sparse_core_info()
    nc, ns = info.num_cores, info.num_subcores
    nw = nc * ns
    b_per_w = S // nw
    mesh = plsc.VectorSubcoreMesh(core_axis_name="c", subcore_axis_name="s")

    @functools.partial(
        pl.kernel, mesh=mesh,
        out_type=jax.ShapeDtypeStruct((S, E), jnp.float32),
        scratch_types=[
            pltpu.VMEM((b_per_w,), jnp.int32),
            pltpu.VMEM((b_per_w, E), jnp.float32),
            pltpu.SemaphoreType.DMA,
        ],
    )
    def k(table_hbm, idx_hbm, out_hbm, idx_v, rows_v, sem):
        wid = lax.axis_index("s") * nc + lax.axis_index("c")
        base = wid * b_per_w
        pltpu.sync_copy(idx_hbm.at[pl.ds(base, b_per_w)], idx_v)
        pltpu.async_copy(table_hbm.at[idx_v], rows_v, sem).wait()
        pltpu.sync_copy(rows_v, out_hbm.at[pl.ds(base, b_per_w)])

    return k


_sc_gather = None


def _gather_rows(emb, ids):
    global _sc_gather
    if _sc_gather is None:
        _sc_gather = _make_sc_gather()
    return _sc_gather(emb, ids)


def _qkv_body(x_ref, g_ref, b_ref, wq_ref, wk_ref, wv_ref,
              bq_ref, bk_ref, bv_ref, q_ref, k_ref, v_ref):
    xn = _bf(_ln(x_ref[...], g_ref[...], b_ref[...]))
    scale = 1.0 / math.sqrt(DH)
    q_ref[...] = _bf((_dot(xn, _bf(wq_ref[...])) + bq_ref[...]) * scale)
    k_ref[...] = _bf(_dot(xn, _bf(wk_ref[...])) + bk_ref[...])
    v_ref[...] = _bf(_dot(xn, _bf(wv_ref[...])) + bv_ref[...])


def _qkv(x, g, b, wq, wk, wv, bq, bk, bv):
    grid = (S // BM,)
    wspec = pl.BlockSpec((D, D), lambda i: (0, 0))
    bspec = pl.BlockSpec((1, D), lambda i: (0, 0))
    ospec = pl.BlockSpec((BM, D), lambda i: (i, 0))
    osh = jax.ShapeDtypeStruct((S, D), jnp.bfloat16)
    return pl.pallas_call(
        _qkv_body,
        grid=grid,
        in_specs=[pl.BlockSpec((BM, D), lambda i: (i, 0)),
                  bspec, bspec, wspec, wspec, wspec, bspec, bspec, bspec],
        out_specs=[ospec, ospec, ospec],
        out_shape=[osh, osh, osh],
        compiler_params=pltpu.CompilerParams(
            dimension_semantics=("parallel",)),
    )(x, g, b, wq, wk, wv, bq, bk, bv)


def _attn_body(q_ref, k_ref, v_ref, o_ref, *, hp):
    outs = []
    for h in range(hp):
        q = q_ref[:, h * DH:(h + 1) * DH]
        k = k_ref[:, h * DH:(h + 1) * DH]
        s = jax.lax.dot_general(
            q, k, (((1,), (1,)), ((), ())),
            preferred_element_type=jnp.float32)
        m = jnp.max(s, axis=-1, keepdims=True)
        e = jnp.exp(s - m)
        r = 1.0 / jnp.sum(e, axis=-1, keepdims=True)
        acc = _dot(_bf(e), v_ref[:, h * DH:(h + 1) * DH])
        outs.append(acc * r)
    o_ref[...] = _bf(jnp.concatenate(outs, axis=1))


def _attention(q, k, v_ext, bq_blk, hp=2):
    grid = (H // hp, S // bq_blk)
    return pl.pallas_call(
        functools.partial(_attn_body, hp=hp),
        grid=grid,
        in_specs=[
            pl.BlockSpec((bq_blk, hp * DH), lambda h, i: (i, h)),
            pl.BlockSpec((S, hp * DH), lambda h, i: (0, h)),
            pl.BlockSpec((S, hp * DH), lambda h, i: (0, h)),
        ],
        out_specs=pl.BlockSpec((bq_blk, hp * DH), lambda h, i: (i, h)),
        out_shape=jax.ShapeDtypeStruct((S, D), jnp.bfloat16),
        compiler_params=pltpu.CompilerParams(
            dimension_semantics=("parallel", "parallel")),
    )(q, k, v_ext)


def _matmul_res_body(x_ref, w_ref, bias_ref, r_ref, o_ref):
    o_ref[...] = _dot(x_ref[...], _bf(w_ref[...])) + bias_ref[...] + r_ref[...]


def _matmul_res(x, w, bias, res, bn):
    k, n = w.shape
    grid = (n // bn, S // BM)
    return pl.pallas_call(
        _matmul_res_body,
        grid=grid,
        in_specs=[
            pl.BlockSpec((BM, k), lambda j, i: (i, 0)),
            pl.BlockSpec((k, bn), lambda j, i: (0, j)),
            pl.BlockSpec((1, bn), lambda j, i: (0, j)),
            pl.BlockSpec((BM, bn), lambda j, i: (i, j)),
        ],
        out_specs=pl.BlockSpec((BM, bn), lambda j, i: (i, j)),
        out_shape=jax.ShapeDtypeStruct((S, n), jnp.float32),
        compiler_params=pltpu.CompilerParams(
            dimension_semantics=("parallel", "parallel")),
    )(x, w, bias, res)


BQA = 256


def _attnblk_body(q_ref, k_ref, v_ref, wo_ref, bo_ref, r_ref, o_ref, o_s):
    for h in range(H):
        qb = q_ref[:, h * DH:(h + 1) * DH]
        s = jax.lax.dot_general(
            qb, k_ref[:, h * DH:(h + 1) * DH], (((1,), (1,)), ((), ())),
            preferred_element_type=jnp.float32)
        m = jnp.max(s, axis=-1, keepdims=True)
        e = jnp.exp(s - m)
        r = 1.0 / jnp.sum(e, axis=-1, keepdims=True)
        o_s[:, h * DH:(h + 1) * DH] = _bf(
            _dot(_bf(e), v_ref[:, h * DH:(h + 1) * DH]) * r)
    o_ref[...] = _dot(o_s[...], _bf(wo_ref[...])) + bo_ref[...] + r_ref[...]


def _attn_block(q, k, v, wo, bov, res):
    grid = (S // BQA,)
    return pl.pallas_call(
        _attnblk_body,
        grid=grid,
        in_specs=[
            pl.BlockSpec((BQA, D), lambda qi: (qi, 0)),
            pl.BlockSpec((S, D), lambda qi: (0, 0)),
            pl.BlockSpec((S, D), lambda qi: (0, 0)),
            pl.BlockSpec((D, D), lambda qi: (0, 0)),
            pl.BlockSpec((1, D), lambda qi: (0, 0)),
            pl.BlockSpec((BQA, D), lambda qi: (qi, 0)),
        ],
        out_specs=pl.BlockSpec((BQA, D), lambda qi: (qi, 0)),
        out_shape=jax.ShapeDtypeStruct((S, D), jnp.float32),
        scratch_shapes=[pltpu.VMEM((BQA, D), jnp.bfloat16)],
        compiler_params=pltpu.CompilerParams(
            dimension_semantics=("parallel",)),
    )(q, k, v, wo, bov, res)


def _ffn_body(x_ref, g_ref, b_ref, w1_ref, b1_ref, w2_ref, b2_ref, o_ref,
              xn_ref):
    j = pl.program_id(0)

    @pl.when(j == 0)
    def _():
        xn_ref[...] = _bf(_ln(x_ref[...], g_ref[...], b_ref[...]))

    a = _dot(xn_ref[...], _bf(w1_ref[...])) + b1_ref[...]
    hb = _bf(0.5 * a * (1.0 + lax.erf(a * (1.0 / math.sqrt(2.0)))))
    contrib = _dot(hb, _bf(w2_ref[...]))

    @pl.when(j == 0)
    def _():
        o_ref[...] = contrib + b2_ref[...] + x_ref[...]

    @pl.when(j > 0)
    def _():
        o_ref[...] += contrib


def _ffn(x, g, b, w1, b1, w2, b2, bf):
    grid = (F // bf,)
    return pl.pallas_call(
        _ffn_body,
        grid=grid,
        in_specs=[
            pl.BlockSpec((S, D), lambda j: (0, 0)),
            pl.BlockSpec((1, D), lambda j: (0, 0)),
            pl.BlockSpec((1, D), lambda j: (0, 0)),
            pl.BlockSpec((D, bf), lambda j: (0, j)),
            pl.BlockSpec((1, bf), lambda j: (0, j)),
            pl.BlockSpec((bf, D), lambda j: (j, 0)),
            pl.BlockSpec((1, D), lambda j: (0, 0)),
        ],
        out_specs=pl.BlockSpec((S, D), lambda j: (0, 0)),
        out_shape=jax.ShapeDtypeStruct((S, D), jnp.float32),
        scratch_shapes=[pltpu.VMEM((S, D), jnp.bfloat16)],
        compiler_params=pltpu.CompilerParams(
            dimension_semantics=("arbitrary",)),
    )(x, g, b, w1, b1, w2, b2)


def _ln_matmul_body(x_ref, g_ref, b_ref, w_ref, bias_ref, o_ref, *, gelu):
    xn = _bf(_ln(x_ref[...], g_ref[...], b_ref[...]))
    acc = _dot(xn, _bf(w_ref[...])) + bias_ref[...]
    if gelu:
        acc = 0.5 * acc * (1.0 + lax.erf(acc * (1.0 / math.sqrt(2.0))))
    o_ref[...] = acc.astype(o_ref.dtype)


def _ln_matmul(x, g, b, w, bias, bn, gelu=False):
    k, n = w.shape
    grid = (n // bn, S // BM)
    return pl.pallas_call(
        functools.partial(_ln_matmul_body, gelu=gelu),
        grid=grid,
        in_specs=[
            pl.BlockSpec((BM, k), lambda j, i: (i, 0)),
            pl.BlockSpec((1, k), lambda j, i: (0, 0)),
            pl.BlockSpec((1, k), lambda j, i: (0, 0)),
            pl.BlockSpec((k, bn), lambda j, i: (0, j)),
            pl.BlockSpec((1, bn), lambda j, i: (0, j)),
        ],
        out_specs=pl.BlockSpec((BM, bn), lambda j, i: (i, j)),
        out_shape=jax.ShapeDtypeStruct((S, n), jnp.bfloat16),
        compiler_params=pltpu.CompilerParams(
            dimension_semantics=("parallel", "parallel")),
    )(x, g, b, w, bias)


def _final_body(x_ref, w_ref, b_ref, o_ref):
    logits = _dot(_bf(x_ref[...]), _bf(w_ref[...])) + b_ref[...]
    m = jnp.max(logits, axis=-1, keepdims=True)
    e = jnp.exp(logits - m)
    o_ref[...] = e / jnp.sum(e, axis=-1, keepdims=True)


def _final(x, w, bias):
    grid = (S // BM,)
    return pl.pallas_call(
        _final_body,
        grid=grid,
        in_specs=[
            pl.BlockSpec((BM, D), lambda i: (i, 0)),
            pl.BlockSpec((D, O), lambda i: (0, 0)),
            pl.BlockSpec((1, O), lambda i: (0, 0)),
        ],
        out_specs=pl.BlockSpec((BM, O), lambda i: (i, 0)),
        out_shape=jax.ShapeDtypeStruct((S, O), jnp.float32),
        compiler_params=pltpu.CompilerParams(
            dimension_semantics=("parallel",)),
    )(x, w, bias)


def kernel(input_ids, probs, emb, ln1_g, ln1_b, Wq, bq, Wk, bk, Wv, bv,
           Wo, bo, ln2_g, ln2_b, W1, b1, W2, b2, Wlast, blast):
    rows = _gather_rows(emb, input_ids.reshape(S))
    x = jnp.concatenate([rows, probs.reshape(S, P)], axis=-1)

    for i in range(L):
        q, k, v = _qkv(x, ln1_g[i].reshape(1, D), ln1_b[i].reshape(1, D),
                       Wq[i], Wk[i], Wv[i], bq[i].reshape(1, D),
                       bk[i].reshape(1, D), bv[i].reshape(1, D))
        o = _attention(q, k, v, bq_blk=512, hp=8)
        x = _matmul_res(o, Wo[i], bo[i].reshape(1, D), x, bn=1024)
        x = _ffn(x, ln2_g[i].reshape(1, D), ln2_b[i].reshape(1, D),
                 W1[i], b1[i].reshape(1, F), W2[i], b2[i].reshape(1, D),
                 bf=1024)

    out = _final(x, Wlast, blast.reshape(1, O))
    return out.reshape(B, S, O)
